# baseline (speedup 1.0000x reference)
import functools

import jax
import jax.numpy as jnp
from jax import lax
from jax.experimental import pallas as pl
from jax.experimental.pallas import tpu as pltpu
from jax.experimental.pallas import tpu_sc as plsc

NC = 2
NS = 16
NW = NC * NS

IDX_W = 128
STREAMS_PER_CHUNK = 10
CHUNK = IDX_W * STREAMS_PER_CHUNK


def _gather_kernel(n_per_w, n_chunks, out_dim,
                   table_hbm, idx_hbm, out_hbm,
                   idx_v, rows_a, rows_b, gsem, osem_a, osem_b):
    wid = lax.axis_index("s") * NC + lax.axis_index("c")

    pltpu.sync_copy(idx_hbm.at[wid], idx_v)

    bufs = (rows_a, rows_b)
    osems = (osem_a, osem_b)

    def do_chunk(c, buf, osem, wait_out):
        copies = []
        for j in range(STREAMS_PER_CHUNK):
            copies.append(pltpu.async_copy(
                table_hbm.at[idx_v.at[c * STREAMS_PER_CHUNK + j]],
                buf.at[pl.ds(j * IDX_W, IDX_W)],
                gsem))
        if wait_out is not None:
            wait_out.wait()
        for cp in copies:
            cp.wait()
        return pltpu.async_copy(
            buf,
            out_hbm.at[pl.ds(wid * n_per_w + c * CHUNK, CHUNK)],
            osem)

    pending = [None, None]
    for c in range(n_chunks):
        p = c % 2
        pending[p] = do_chunk(c, bufs[p], osems[p], pending[p])
    for cp in pending:
        if cp is not None:
            cp.wait()


def kernel(inputs, embeddings):
    batch, hist = inputs.shape
    in_dim, out_dim = embeddings.shape
    n_total = batch * hist
    n_per_w = n_total // NW
    assert n_per_w * NW == n_total
    assert n_per_w % CHUNK == 0
    n_chunks = n_per_w // CHUNK

    idx = inputs.reshape(NW, n_per_w // IDX_W, IDX_W)

    mesh = plsc.VectorSubcoreMesh(
        core_axis_name="c", subcore_axis_name="s",
        num_cores=NC, num_subcores=NS)

    grab = pl.kernel(
        functools.partial(_gather_kernel, n_per_w, n_chunks, out_dim),
        out_type=jax.ShapeDtypeStruct((n_total, out_dim), jnp.float32),
        mesh=mesh,
        scratch_types=[
            pltpu.VMEM((n_per_w // IDX_W, IDX_W), jnp.int32),
            pltpu.VMEM((CHUNK, out_dim), jnp.float32),
            pltpu.VMEM((CHUNK, out_dim), jnp.float32),
            pltpu.SemaphoreType.DMA,
            pltpu.SemaphoreType.DMA,
            pltpu.SemaphoreType.DMA,
        ],
        compiler_params=pltpu.CompilerParams(use_tc_tiling_on_sc=False),
    )
    out = grab(embeddings, idx)
    return out.reshape(batch, hist, out_dim)


# speedup vs baseline: 1.2287x; 1.2287x over previous
import functools

import jax
import jax.numpy as jnp
from jax import lax
from jax.experimental import pallas as pl
from jax.experimental.pallas import tpu as pltpu
from jax.experimental.pallas import tpu_sc as plsc

NC = 2
NS = 16
NW = NC * NS

BLK = 128
HSLOT = 5


def _gather_kernel(hist, table_hbm, idx_hbm, out_hbm,
                   idx_v, rows_a, rows_b, gsem_a, gsem_b, osem_a, osem_b):
    wid = lax.axis_index("s") * NC + lax.axis_index("c")
    b0 = wid * BLK

    pltpu.sync_copy(idx_hbm.at[:, pl.ds(b0, BLK)], idx_v)

    n_chunks = hist // HSLOT
    bufs = (rows_a, rows_b)
    gsems = (gsem_a, gsem_b)
    osems = (osem_a, osem_b)

    pend_out = [None, None]
    for c in range(n_chunks):
        p = c % 2
        buf, gsem, osem = bufs[p], gsems[p], osems[p]
        if pend_out[p] is not None:
            for cp in pend_out[p]:
                cp.wait()
        gathers = []
        for s in range(HSLOT):
            h = c * HSLOT + s
            gathers.append(pltpu.async_copy(
                table_hbm.at[idx_v.at[h]],
                buf.at[s],
                gsem))
        writes = []
        for s in range(HSLOT):
            h = c * HSLOT + s
            gathers[s].wait()
            writes.append(pltpu.async_copy(
                buf.at[s],
                out_hbm.at[pl.ds(b0, BLK), h],
                osem))
        pend_out[p] = writes
    for writes in pend_out:
        if writes is not None:
            for cp in writes:
                cp.wait()


def kernel(inputs, embeddings):
    batch, hist = inputs.shape
    in_dim, out_dim = embeddings.shape
    assert batch == NW * BLK
    assert hist % HSLOT == 0

    idx_t = inputs.T

    mesh = plsc.VectorSubcoreMesh(
        core_axis_name="c", subcore_axis_name="s",
        num_cores=NC, num_subcores=NS)

    grab = pl.kernel(
        functools.partial(_gather_kernel, hist),
        out_type=jax.ShapeDtypeStruct((batch, hist, out_dim), jnp.float32),
        mesh=mesh,
        scratch_types=[
            pltpu.VMEM((hist, BLK), jnp.int32),
            pltpu.VMEM((HSLOT, BLK, out_dim), jnp.float32),
            pltpu.VMEM((HSLOT, BLK, out_dim), jnp.float32),
            pltpu.SemaphoreType.DMA,
            pltpu.SemaphoreType.DMA,
            pltpu.SemaphoreType.DMA,
            pltpu.SemaphoreType.DMA,
        ],
        compiler_params=pltpu.CompilerParams(use_tc_tiling_on_sc=False),
    )
    return grab(embeddings, idx_t)


# speedup vs baseline: 1.4581x; 1.1868x over previous
import functools

import jax
import jax.numpy as jnp
from jax import lax
from jax.experimental import pallas as pl
from jax.experimental.pallas import tpu as pltpu
from jax.experimental.pallas import tpu_sc as plsc

NC = 2
NS = 16
NW = NC * NS

BLK = 128
HSLOT = 5

TCOLS = 8192


def _relayout_kernel(in_ref, out_ref):
    tr = in_ref[...].T.reshape(TCOLS // 4, 4, 32)
    out_ref[...] = jnp.concatenate([tr[:, m, :] for m in range(4)], axis=1)


def _gather_kernel(hist, table_hbm, idx_hbm, out_hbm,
                   idx_v, rows_a, rows_b, gsem_a, gsem_b, osem_a, osem_b):
    wid = lax.axis_index("s") * NC + lax.axis_index("c")
    b0 = wid * BLK

    pltpu.sync_copy(idx_hbm.at[:, pl.ds(b0, BLK)], idx_v)

    n_chunks = hist // HSLOT
    bufs = (rows_a, rows_b)
    gsems = (gsem_a, gsem_b)
    osems = (osem_a, osem_b)

    pend_out = [None, None]
    for c in range(n_chunks):
        p = c % 2
        buf, gsem, osem = bufs[p], gsems[p], osems[p]
        if pend_out[p] is not None:
            for cp in pend_out[p]:
                cp.wait()
        gathers = []
        for s in range(HSLOT):
            h = c * HSLOT + s
            gathers.append(pltpu.async_copy(
                table_hbm.at[idx_v.at[h]],
                buf.at[s],
                gsem))
        writes = []
        for s in range(HSLOT):
            h = c * HSLOT + s
            gathers[s].wait()
            writes.append(pltpu.async_copy(
                buf.at[s],
                out_hbm.at[pl.ds(b0, BLK), h],
                osem))
        pend_out[p] = writes
    for writes in pend_out:
        if writes is not None:
            for cp in writes:
                cp.wait()


def kernel(inputs, embeddings):
    batch, hist = inputs.shape
    in_dim, out_dim = embeddings.shape
    assert batch == NW * BLK
    assert hist % HSLOT == 0
    assert (TCOLS * out_dim) % 128 == 0

    emb_t = embeddings.T
    n_blocks = -(-in_dim // TCOLS)
    lin = pl.pallas_call(
        _relayout_kernel,
        grid=(n_blocks,),
        in_specs=[pl.BlockSpec((out_dim, TCOLS), lambda i: (0, i))],
        out_specs=pl.BlockSpec((TCOLS // 4, 128), lambda i: (i, 0)),
        out_shape=jax.ShapeDtypeStruct((in_dim * out_dim // 128, 128),
                                       jnp.float32),
    )(emb_t)
    table = lin.reshape(in_dim, out_dim)

    idx_t = inputs.T

    mesh = plsc.VectorSubcoreMesh(
        core_axis_name="c", subcore_axis_name="s",
        num_cores=NC, num_subcores=NS)

    grab = pl.kernel(
        functools.partial(_gather_kernel, hist),
        out_type=jax.ShapeDtypeStruct((batch, hist, out_dim), jnp.float32),
        mesh=mesh,
        scratch_types=[
            pltpu.VMEM((hist, BLK), jnp.int32),
            pltpu.VMEM((HSLOT, BLK, out_dim), jnp.float32),
            pltpu.VMEM((HSLOT, BLK, out_dim), jnp.float32),
            pltpu.SemaphoreType.DMA,
            pltpu.SemaphoreType.DMA,
            pltpu.SemaphoreType.DMA,
            pltpu.SemaphoreType.DMA,
        ],
        compiler_params=pltpu.CompilerParams(use_tc_tiling_on_sc=False),
    )
    return grab(table, idx_t)
